# baseline (speedup 1.0000x reference)
import functools

import jax
import jax.numpy as jnp
from jax import lax
from jax.experimental import pallas as pl
from jax.experimental.pallas import tpu as pltpu
from jax.experimental.pallas import tpu_sc as plsc

_NC = 2
_NS = 16
_NW = _NC * _NS

_CH = 32
_NBUF = 3
_LOOKAHEAD = _NBUF - 1


def _sc_body(emb_hbm, out_hbm, buf, lsem, ssem, *, rows_per_w, dim):
    cid = lax.axis_index("c")
    sid = lax.axis_index("s")
    wid = sid * _NC + cid
    base = wid * rows_per_w
    nchunks = rows_per_w // _CH

    def load(i):
        pltpu.async_copy(
            emb_hbm.at[pl.ds(base + i * _CH, _CH)], buf.at[i % _NBUF],
            lsem.at[i % _NBUF])

    def store_pair(i):
        return (buf.at[i % _NBUF],
                out_hbm.at[pl.ds(base + i * _CH, _CH)], ssem.at[i % _NBUF])

    for i in range(_LOOKAHEAD):
        load(i)
    for i in range(nchunks):
        pltpu.make_async_copy(
            emb_hbm.at[pl.ds(base + i * _CH, _CH)], buf.at[i % _NBUF],
            lsem.at[i % _NBUF]).wait()
        pltpu.async_copy(*store_pair(i))
        ni = i + _LOOKAHEAD
        if ni < nchunks:
            prev = ni - _NBUF
            if prev >= 0:
                pltpu.make_async_copy(*store_pair(prev)).wait()
            load(ni)
    for i in range(max(0, nchunks - _NBUF), nchunks):
        pltpu.make_async_copy(*store_pair(i)).wait()


def kernel(x, emb):
    seq_len = x.shape[1]
    dim = emb.shape[1]
    rows_per_w = seq_len // _NW
    mesh = plsc.VectorSubcoreMesh(core_axis_name="c", subcore_axis_name="s")
    body = functools.partial(_sc_body, rows_per_w=rows_per_w, dim=dim)
    return pl.kernel(
        body,
        out_type=jax.ShapeDtypeStruct((seq_len, dim), emb.dtype),
        mesh=mesh,
        scratch_types=[
            pltpu.VMEM((_NBUF, _CH, dim), emb.dtype),
            pltpu.SemaphoreType.DMA((_NBUF,)),
            pltpu.SemaphoreType.DMA((_NBUF,)),
        ],
    )(emb[:seq_len])


# speedup vs baseline: 1.0374x; 1.0374x over previous
import functools

import jax
import jax.numpy as jnp
from jax import lax
from jax.experimental import pallas as pl
from jax.experimental.pallas import tpu as pltpu
from jax.experimental.pallas import tpu_sc as plsc

_NC = 2
_NS = 16
_NW = _NC * _NS

_CH = 16
_NBUF = 7
_LOOKAHEAD = _NBUF - 1


def _sc_body(emb_hbm, out_hbm, buf, lsem, ssem, *, rows_per_w, dim):
    cid = lax.axis_index("c")
    sid = lax.axis_index("s")
    wid = sid * _NC + cid
    base = wid * rows_per_w
    nchunks = rows_per_w // _CH

    def load(i):
        pltpu.async_copy(
            emb_hbm.at[pl.ds(base + i * _CH, _CH)], buf.at[i % _NBUF],
            lsem.at[i % _NBUF])

    def store_pair(i):
        return (buf.at[i % _NBUF],
                out_hbm.at[pl.ds(base + i * _CH, _CH)], ssem.at[i % _NBUF])

    for i in range(_LOOKAHEAD):
        load(i)
    for i in range(nchunks):
        pltpu.make_async_copy(
            emb_hbm.at[pl.ds(base + i * _CH, _CH)], buf.at[i % _NBUF],
            lsem.at[i % _NBUF]).wait()
        pltpu.async_copy(*store_pair(i))
        ni = i + _LOOKAHEAD
        if ni < nchunks:
            prev = ni - _NBUF
            if prev >= 0:
                pltpu.make_async_copy(*store_pair(prev)).wait()
            load(ni)
    for i in range(max(0, nchunks - _NBUF), nchunks):
        pltpu.make_async_copy(*store_pair(i)).wait()


def kernel(x, emb):
    seq_len = x.shape[1]
    dim = emb.shape[1]
    rows_per_w = seq_len // _NW
    mesh = plsc.VectorSubcoreMesh(core_axis_name="c", subcore_axis_name="s")
    body = functools.partial(_sc_body, rows_per_w=rows_per_w, dim=dim)
    return pl.kernel(
        body,
        out_type=jax.ShapeDtypeStruct((seq_len, dim), emb.dtype),
        mesh=mesh,
        scratch_types=[
            pltpu.VMEM((_NBUF, _CH, dim), emb.dtype),
            pltpu.SemaphoreType.DMA((_NBUF,)),
            pltpu.SemaphoreType.DMA((_NBUF,)),
        ],
    )(emb[:seq_len])
